# baseline (device time: 155423 ns/iter reference)
import jax
import jax.numpy as jnp
from jax import lax
from jax.experimental import pallas as pl
from jax.experimental.pallas import tpu as pltpu

N_DEV = 4
SQ = 1024
DM = 1024
GW = 1024
HG = 8
DH = 128
NR = 4
RB = SQ // NR
NB = 16
SCALE = 0.08838834764831843


def _perm_rows(a):
    c = a.shape[1]
    return a.reshape(4, 4, 64, c).swapaxes(0, 1).reshape(SQ, c)


def kernel(x, Wq, K_ext, V_ext, Wo):
    xb = _perm_rows((x[0] * SCALE).astype(jnp.bfloat16))
    wq = Wq.astype(jnp.bfloat16)
    wo = Wo.astype(jnp.bfloat16)

    def body(x_ref, wq_ref, wo_ref, k_hbm, v_hbm, out_ref,
             comm_wq, comm_wo, kbuf, vbuf, qbuf, ctx, khead, vhead,
             sq_send, sq_recv, so_send, so_recv, kv_sems):
        me = lax.axis_index("i")
        right = lax.rem(me + 1, N_DEV)
        left = lax.rem(me + 3, N_DEV)

        barrier_sem = pltpu.get_barrier_semaphore()
        for nbr in (left, right):
            pl.semaphore_signal(
                barrier_sem, inc=1,
                device_id=(nbr,), device_id_type=pl.DeviceIdType.MESH,
            )
        pl.semaphore_wait(barrier_sem, 2)

        def hop(h):
            dq = pltpu.make_async_remote_copy(
                src_ref=wq_ref if h == 1 else comm_wq.at[h - 2],
                dst_ref=comm_wq.at[h - 1],
                send_sem=sq_send.at[h - 1],
                recv_sem=sq_recv.at[h - 1],
                device_id=(right,),
                device_id_type=pl.DeviceIdType.MESH,
            )
            do = pltpu.make_async_remote_copy(
                src_ref=wo_ref if h == 1 else comm_wo.at[h - 2],
                dst_ref=comm_wo.at[h - 1],
                send_sem=so_send.at[h - 1],
                recv_sem=so_recv.at[h - 1],
                device_id=(left,),
                device_id_type=pl.DeviceIdType.MESH,
            )
            dq.start()
            do.start()
            return dq, do

        def kv_load(src, buf, p):
            g = lax.rem(me - p + N_DEV, N_DEV)
            slot = p % 2
            sem = kv_sems.at[0 if src is k_hbm else 1, slot]
            ds = []
            for b in range(NB):
                qi, r = b // 4, b % 4
                d0 = r * RB + qi * 64
                d = pltpu.make_async_copy(
                    src.at[me, pl.ds(b * 64, 64), pl.ds(g * HG, HG), :],
                    buf.at[slot, pl.ds(d0, 64), :, :],
                    sem,
                )
                d.start()
                ds.append(d)
            return ds

        def wq_at(p):
            return wq_ref[:, :] if p == 0 else comm_wq[p - 1, :, :]

        def wo_at(q):
            return wo_ref[:, :] if q == 0 else comm_wo[q - 1, :, :]

        def attn(p):
            slot = p % 2
            qbuf[:, :] = jnp.dot(
                x_ref[:, :], wq_at(p), preferred_element_type=jnp.float32
            ).astype(jnp.bfloat16)
            c0 = p * GW
            for h in range(HG):
                qc = h * DH
                kc = c0 + h * DH
                khead[:, :] = kbuf[slot, :, h, :].astype(jnp.bfloat16)
                vhead[:, :] = vbuf[slot, :, h, :].astype(jnp.bfloat16)

                def r_body(r, _, qc=qc, kc=kc):
                    row = r * RB
                    q = qbuf[pl.ds(row, RB), qc:qc + DH]
                    k = khead[pl.ds(row, RB), :]
                    s = lax.dot_general(
                        q, k, (((1,), (1,)), ((), ())),
                        preferred_element_type=jnp.float32,
                    )
                    m = jnp.max(s, axis=1, keepdims=True)
                    e = jnp.exp(s - m)
                    w = (e / jnp.sum(e, axis=1, keepdims=True)).astype(jnp.bfloat16)
                    v = vhead[pl.ds(row, RB), :]
                    ctx[pl.ds(row, RB), kc:kc + DH] = jnp.dot(
                        w, v, preferred_element_type=jnp.float32
                    ).astype(jnp.bfloat16)
                    return 0

                lax.fori_loop(0, NR, r_body, 0)

        def outproj(q, first=False):
            pblk = (N_DEV - q) % N_DEV
            contrib = jnp.dot(
                ctx[:, pblk * GW:(pblk + 1) * GW], wo_at(q),
                preferred_element_type=jnp.float32,
            )
            if first:
                out_ref[:, :] = contrib
            else:
                out_ref[:, :] = out_ref[:, :] + contrib

        dq1, do1 = hop(1)
        dk = kv_load(k_hbm, kbuf, 0)
        dv = kv_load(v_hbm, vbuf, 0)
        for d in dk + dv:
            d.wait()
        dk = kv_load(k_hbm, kbuf, 1)
        dv = kv_load(v_hbm, vbuf, 1)

        attn(0)
        outproj(0, first=True)

        dq1.wait_recv()
        do1.wait_recv()
        dq2, do2 = hop(2)
        for d in dk + dv:
            d.wait()
        dk = kv_load(k_hbm, kbuf, 2)
        dv = kv_load(v_hbm, vbuf, 2)

        attn(1)

        dq2.wait_recv()
        do2.wait_recv()
        dq3, do3 = hop(3)
        for d in dk + dv:
            d.wait()
        dk = kv_load(k_hbm, kbuf, 3)
        dv = kv_load(v_hbm, vbuf, 3)

        attn(2)
        outproj(2)

        dq3.wait_recv()
        do3.wait_recv()
        for d in dk + dv:
            d.wait()

        attn(3)
        outproj(1)
        outproj(3)

        for d in (dq1, do1, dq2, do2, dq3, do3):
            d.wait_send()

    out_p = pl.pallas_call(
        body,
        out_shape=jax.ShapeDtypeStruct((SQ, DM), jnp.float32),
        in_specs=[
            pl.BlockSpec(memory_space=pltpu.VMEM),
            pl.BlockSpec(memory_space=pltpu.VMEM),
            pl.BlockSpec(memory_space=pltpu.VMEM),
            pl.BlockSpec(memory_space=pltpu.MemorySpace.HBM),
            pl.BlockSpec(memory_space=pltpu.MemorySpace.HBM),
        ],
        out_specs=pl.BlockSpec(memory_space=pltpu.VMEM),
        scratch_shapes=[
            pltpu.VMEM((3, DM, GW), jnp.bfloat16),
            pltpu.VMEM((3, GW, DM), jnp.bfloat16),
            pltpu.VMEM((2, SQ, HG, DH), jnp.float32),
            pltpu.VMEM((2, SQ, HG, DH), jnp.float32),
            pltpu.VMEM((SQ, GW), jnp.bfloat16),
            pltpu.VMEM((SQ, N_DEV * GW), jnp.bfloat16),
            pltpu.VMEM((SQ, DH), jnp.bfloat16),
            pltpu.VMEM((SQ, DH), jnp.bfloat16),
            pltpu.SemaphoreType.DMA((3,)),
            pltpu.SemaphoreType.DMA((3,)),
            pltpu.SemaphoreType.DMA((3,)),
            pltpu.SemaphoreType.DMA((3,)),
            pltpu.SemaphoreType.DMA((2, 2)),
        ],
        compiler_params=pltpu.CompilerParams(
            collective_id=0,
            vmem_limit_bytes=62 * 1024 * 1024,
        ),
    )(xb, wq, wo, K_ext, V_ext)

    return _perm_rows(out_p)[None]


# device time: 112879 ns/iter; 1.3769x vs baseline; 1.3769x over previous
import jax
import jax.numpy as jnp
from jax import lax
from jax.experimental import pallas as pl
from jax.experimental.pallas import tpu as pltpu

N_DEV = 4
SQ = 1024
DM = 1024
GW = 1024
HG = 8
DH = 128
NR = 4
RB = SQ // NR
NB = 16
SCALE = 0.08838834764831843


def _perm_rows(a):
    c = a.shape[1]
    return a.reshape(4, 4, 64, c).swapaxes(0, 1).reshape(SQ, c)


def kernel(x, Wq, K_ext, V_ext, Wo):
    xb = _perm_rows((x[0] * SCALE).astype(jnp.bfloat16))
    wq = Wq.astype(jnp.bfloat16)
    wo = Wo.astype(jnp.bfloat16)

    def body(x_ref, wq_ref, wo_ref, k_hbm, v_hbm, out_ref,
             comm_wq, comm_wo, kbuf, vbuf, qbuf, ctx,
             sq_send, sq_recv, so_send, so_recv, kv_sems):
        me = lax.axis_index("i")
        right = lax.rem(me + 1, N_DEV)
        left = lax.rem(me + 3, N_DEV)

        barrier_sem = pltpu.get_barrier_semaphore()
        for nbr in (left, right):
            pl.semaphore_signal(
                barrier_sem, inc=1,
                device_id=(nbr,), device_id_type=pl.DeviceIdType.MESH,
            )
        pl.semaphore_wait(barrier_sem, 2)

        def hop(h):
            dq = pltpu.make_async_remote_copy(
                src_ref=wq_ref if h == 1 else comm_wq.at[h - 2],
                dst_ref=comm_wq.at[h - 1],
                send_sem=sq_send.at[h - 1],
                recv_sem=sq_recv.at[h - 1],
                device_id=(right,),
                device_id_type=pl.DeviceIdType.MESH,
            )
            do = pltpu.make_async_remote_copy(
                src_ref=wo_ref if h == 1 else comm_wo.at[h - 2],
                dst_ref=comm_wo.at[h - 1],
                send_sem=so_send.at[h - 1],
                recv_sem=so_recv.at[h - 1],
                device_id=(left,),
                device_id_type=pl.DeviceIdType.MESH,
            )
            dq.start()
            do.start()
            return dq, do

        def kv_load(src, buf, p):
            g = lax.rem(me - p + N_DEV, N_DEV)
            slot = p % 2
            sem = kv_sems.at[0 if src is k_hbm else 1, slot]
            ds = []
            for b in range(NB):
                qi, r = b // 4, b % 4
                d0 = r * RB + qi * 64
                d = pltpu.make_async_copy(
                    src.at[me, pl.ds(b * 64, 64), pl.ds(g * HG, HG), :],
                    buf.at[slot, pl.ds(d0, 64), :, :],
                    sem,
                )
                d.start()
                ds.append(d)
            return ds

        def wq_at(p):
            return wq_ref[:, :] if p == 0 else comm_wq[p - 1, :, :]

        def wo_at(q):
            return wo_ref[:, :] if q == 0 else comm_wo[q - 1, :, :]

        def attn(p):
            slot = p % 2
            qbuf[:, :] = jnp.dot(
                x_ref[:, :], wq_at(p), preferred_element_type=jnp.float32
            ).astype(jnp.bfloat16)
            c0 = p * GW
            for h in range(HG):
                qc = h * DH
                kc = c0 + h * DH
                kh = kbuf[slot, :, h, :].astype(jnp.bfloat16)
                vh = vbuf[slot, :, h, :].astype(jnp.bfloat16)
                for r in range(NR):
                    row = r * RB
                    q = qbuf[row:row + RB, qc:qc + DH]
                    s = lax.dot_general(
                        q, kh[row:row + RB, :], (((1,), (1,)), ((), ())),
                        preferred_element_type=jnp.float32,
                    )
                    m = jnp.max(s, axis=1, keepdims=True)
                    e = jnp.exp(s - m)
                    w = (e / jnp.sum(e, axis=1, keepdims=True)).astype(jnp.bfloat16)
                    ctx[row:row + RB, kc:kc + DH] = jnp.dot(
                        w, vh[row:row + RB, :], preferred_element_type=jnp.float32
                    ).astype(jnp.bfloat16)

        def outproj(q, first=False):
            pblk = (N_DEV - q) % N_DEV
            contrib = jnp.dot(
                ctx[:, pblk * GW:(pblk + 1) * GW], wo_at(q),
                preferred_element_type=jnp.float32,
            )
            if first:
                out_ref[:, :] = contrib
            else:
                out_ref[:, :] = out_ref[:, :] + contrib

        dq1, do1 = hop(1)
        dk = kv_load(k_hbm, kbuf, 0)
        dv = kv_load(v_hbm, vbuf, 0)
        for d in dk + dv:
            d.wait()
        dk = kv_load(k_hbm, kbuf, 1)
        dv = kv_load(v_hbm, vbuf, 1)

        attn(0)
        outproj(0, first=True)

        dq1.wait_recv()
        do1.wait_recv()
        dq2, do2 = hop(2)
        for d in dk + dv:
            d.wait()
        dk = kv_load(k_hbm, kbuf, 2)
        dv = kv_load(v_hbm, vbuf, 2)

        attn(1)

        dq2.wait_recv()
        do2.wait_recv()
        dq3, do3 = hop(3)
        for d in dk + dv:
            d.wait()
        dk = kv_load(k_hbm, kbuf, 3)
        dv = kv_load(v_hbm, vbuf, 3)

        attn(2)
        outproj(2)

        dq3.wait_recv()
        do3.wait_recv()
        for d in dk + dv:
            d.wait()

        attn(3)
        outproj(1)
        outproj(3)

        for d in (dq1, do1, dq2, do2, dq3, do3):
            d.wait_send()

    out_p = pl.pallas_call(
        body,
        out_shape=jax.ShapeDtypeStruct((SQ, DM), jnp.float32),
        in_specs=[
            pl.BlockSpec(memory_space=pltpu.VMEM),
            pl.BlockSpec(memory_space=pltpu.VMEM),
            pl.BlockSpec(memory_space=pltpu.VMEM),
            pl.BlockSpec(memory_space=pltpu.MemorySpace.HBM),
            pl.BlockSpec(memory_space=pltpu.MemorySpace.HBM),
        ],
        out_specs=pl.BlockSpec(memory_space=pltpu.VMEM),
        scratch_shapes=[
            pltpu.VMEM((3, DM, GW), jnp.bfloat16),
            pltpu.VMEM((3, GW, DM), jnp.bfloat16),
            pltpu.VMEM((2, SQ, HG, DH), jnp.float32),
            pltpu.VMEM((2, SQ, HG, DH), jnp.float32),
            pltpu.VMEM((SQ, GW), jnp.bfloat16),
            pltpu.VMEM((SQ, N_DEV * GW), jnp.bfloat16),
            pltpu.SemaphoreType.DMA((3,)),
            pltpu.SemaphoreType.DMA((3,)),
            pltpu.SemaphoreType.DMA((3,)),
            pltpu.SemaphoreType.DMA((3,)),
            pltpu.SemaphoreType.DMA((2, 2)),
        ],
        compiler_params=pltpu.CompilerParams(
            collective_id=0,
            vmem_limit_bytes=62 * 1024 * 1024,
        ),
    )(xb, wq, wo, K_ext, V_ext)

    return _perm_rows(out_p)[None]


# device time: 112875 ns/iter; 1.3769x vs baseline; 1.0000x over previous
import jax
import jax.numpy as jnp
from jax import lax
from jax.experimental import pallas as pl
from jax.experimental.pallas import tpu as pltpu

N_DEV = 4
SQ = 1024
DM = 1024
GW = 1024
HG = 8
DH = 128
NR = 4
RB = SQ // NR
NB = 16
SCALE = 0.08838834764831843


def _perm_rows(a):
    c = a.shape[1]
    return a.reshape(4, 4, 64, c).swapaxes(0, 1).reshape(SQ, c)


def kernel(x, Wq, K_ext, V_ext, Wo):
    xb = _perm_rows((x[0] * SCALE).astype(jnp.bfloat16))
    wq = Wq.astype(jnp.bfloat16)
    wo = Wo.astype(jnp.bfloat16)

    def body(x_ref, wq_ref, wo_ref, k_hbm, v_hbm, out_ref,
             comm_wq, comm_wo, kbuf, vbuf, qbuf, ctx,
             sq_send, sq_recv, so_send, so_recv, kv_sems):
        me = lax.axis_index("i")
        right = lax.rem(me + 1, N_DEV)
        left = lax.rem(me + 3, N_DEV)

        barrier_sem = pltpu.get_barrier_semaphore()
        for nbr in (left, right):
            pl.semaphore_signal(
                barrier_sem, inc=1,
                device_id=(nbr,), device_id_type=pl.DeviceIdType.MESH,
            )
        pl.semaphore_wait(barrier_sem, 2)

        def hop(h):
            dq = pltpu.make_async_remote_copy(
                src_ref=wq_ref if h == 1 else comm_wq.at[h - 2],
                dst_ref=comm_wq.at[h - 1],
                send_sem=sq_send.at[h - 1],
                recv_sem=sq_recv.at[h - 1],
                device_id=(right,),
                device_id_type=pl.DeviceIdType.MESH,
            )
            do = pltpu.make_async_remote_copy(
                src_ref=wo_ref if h == 1 else comm_wo.at[h - 2],
                dst_ref=comm_wo.at[h - 1],
                send_sem=so_send.at[h - 1],
                recv_sem=so_recv.at[h - 1],
                device_id=(left,),
                device_id_type=pl.DeviceIdType.MESH,
            )
            dq.start()
            do.start()
            return dq, do

        def kv_load(src, buf, p):
            g = lax.rem(me - p + N_DEV, N_DEV)
            slot = p % 2
            sem = kv_sems.at[0 if src is k_hbm else 1, slot]
            ds = []
            for b in range(NB):
                qi, r = b // 4, b % 4
                d0 = r * RB + qi * 64
                d = pltpu.make_async_copy(
                    src.at[me, pl.ds(b * 64, 64), pl.ds(g * HG, HG), :],
                    buf.at[slot, pl.ds(d0, 64), :, :],
                    sem,
                )
                d.start()
                ds.append(d)
            return ds

        def wq_at(p):
            return wq_ref[:, :] if p == 0 else comm_wq[p - 1, :, :]

        def wo_at(q):
            return wo_ref[:, :] if q == 0 else comm_wo[q - 1, :, :]

        def attn(p):
            slot = p % 2
            qbuf[:, :] = jnp.dot(
                x_ref[:, :], wq_at(p), preferred_element_type=jnp.float32
            ).astype(jnp.bfloat16)
            c0 = p * GW
            for h in range(HG):
                qc = h * DH
                kc = c0 + h * DH
                kh = kbuf[slot, :, h, :].astype(jnp.bfloat16)
                vh = vbuf[slot, :, h, :].astype(jnp.bfloat16)
                for r in range(NR):
                    row = r * RB
                    q = qbuf[row:row + RB, qc:qc + DH]
                    s = lax.dot_general(
                        q, kh[row:row + RB, :], (((1,), (1,)), ((), ())),
                        preferred_element_type=jnp.float32,
                    )
                    m = jnp.max(s, axis=1, keepdims=True)
                    e = jnp.exp(s - m)
                    w = (e / jnp.sum(e, axis=1, keepdims=True)).astype(jnp.bfloat16)
                    ctx[row:row + RB, kc:kc + DH] = jnp.dot(
                        w, vh[row:row + RB, :], preferred_element_type=jnp.float32
                    ).astype(jnp.bfloat16)

        def outproj(q, first=False):
            pblk = (N_DEV - q) % N_DEV
            contrib = jnp.dot(
                ctx[:, pblk * GW:(pblk + 1) * GW], wo_at(q),
                preferred_element_type=jnp.float32,
            )
            if first:
                out_ref[:, :] = contrib
            else:
                out_ref[:, :] = out_ref[:, :] + contrib

        dq1, do1 = hop(1)
        dk = kv_load(k_hbm, kbuf, 0)
        dv = kv_load(v_hbm, vbuf, 0)
        for d in dk + dv:
            d.wait()
        dk = kv_load(k_hbm, kbuf, 1)
        dv = kv_load(v_hbm, vbuf, 1)

        attn(0)
        outproj(0, first=True)

        for d in dk + dv:
            d.wait()
        dk2 = kv_load(k_hbm, kbuf, 2)
        dv2 = kv_load(v_hbm, vbuf, 2)

        dq1.wait_recv()
        do1.wait_recv()
        dq2, do2 = hop(2)

        attn(1)

        for d in dk2 + dv2:
            d.wait()
        dk3 = kv_load(k_hbm, kbuf, 3)
        dv3 = kv_load(v_hbm, vbuf, 3)

        dq2.wait_recv()
        do2.wait_recv()
        dq3, do3 = hop(3)

        attn(2)
        outproj(2)

        dq3.wait_recv()
        do3.wait_recv()
        for d in dk3 + dv3:
            d.wait()

        outproj(3)
        attn(3)
        outproj(1)

        for d in (dq1, do1, dq2, do2, dq3, do3):
            d.wait_send()

    out_p = pl.pallas_call(
        body,
        out_shape=jax.ShapeDtypeStruct((SQ, DM), jnp.float32),
        in_specs=[
            pl.BlockSpec(memory_space=pltpu.VMEM),
            pl.BlockSpec(memory_space=pltpu.VMEM),
            pl.BlockSpec(memory_space=pltpu.VMEM),
            pl.BlockSpec(memory_space=pltpu.MemorySpace.HBM),
            pl.BlockSpec(memory_space=pltpu.MemorySpace.HBM),
        ],
        out_specs=pl.BlockSpec(memory_space=pltpu.VMEM),
        scratch_shapes=[
            pltpu.VMEM((3, DM, GW), jnp.bfloat16),
            pltpu.VMEM((3, GW, DM), jnp.bfloat16),
            pltpu.VMEM((2, SQ, HG, DH), jnp.float32),
            pltpu.VMEM((2, SQ, HG, DH), jnp.float32),
            pltpu.VMEM((SQ, GW), jnp.bfloat16),
            pltpu.VMEM((SQ, N_DEV * GW), jnp.bfloat16),
            pltpu.SemaphoreType.DMA((3,)),
            pltpu.SemaphoreType.DMA((3,)),
            pltpu.SemaphoreType.DMA((3,)),
            pltpu.SemaphoreType.DMA((3,)),
            pltpu.SemaphoreType.DMA((2, 2)),
        ],
        compiler_params=pltpu.CompilerParams(
            collective_id=0,
            vmem_limit_bytes=62 * 1024 * 1024,
        ),
    )(xb, wq, wo, K_ext, V_ext)

    return _perm_rows(out_p)[None]
